# baseline (speedup 1.0000x reference)
import jax
import jax.numpy as jnp
from jax import lax
from jax.experimental import pallas as pl
from jax.experimental.pallas import tpu as pltpu
from jax.experimental.pallas import tpu_sc as plsc

SEQ = 512
CLS_ID = 101
SEP_ID = 102
PAD_ID = 0
TRIM = SEQ - 2

NC = 2
NS = 16
NW = NC * NS
B = 1024
ROWS_PER_W = B // NW
NQ = 4
QROWS = ROWS_PER_W // NQ


def _words_body(tokens_hbm, lengths_hbm, word_hbm,
                tok_v, word_v, len_v, sems_in, sems_out):
    wid = lax.axis_index("s") * NC + lax.axis_index("c")
    base = wid * ROWS_PER_W

    cins = [
        pltpu.async_copy(tokens_hbm.at[pl.ds(base + q * QROWS, QROWS)],
                         tok_v.at[pl.ds(q * QROWS, QROWS)], sems_in.at[q])
        for q in range(NQ)
    ]
    pltpu.sync_copy(lengths_hbm.at[pl.ds(base, ROWS_PER_W)], len_v)

    iota16 = lax.iota(jnp.int32, 16)
    iota_m1 = iota16 - 1

    def row_body(r, _):
        rvec = jnp.full((16,), r, jnp.int32)
        eff = jnp.minimum(plsc.load_gather(len_v, [rvec]), TRIM)
        eff1 = eff + 1

        g0 = plsc.load_gather(tok_v, [rvec, jnp.maximum(iota_m1, 0)])
        sep0 = jnp.where(iota16 == eff1, jnp.int32(SEP_ID), jnp.int32(PAD_ID))
        word_v[r, pl.ds(0, 16)] = jnp.where(
            iota16 == 0, jnp.int32(CLS_ID),
            jnp.where(iota16 <= eff, g0, sep0))

        for k in range(1, SEQ // 16):
            i = k * 16
            p = iota16 + i
            g = plsc.load_gather(tok_v, [rvec, iota_m1 + i])
            word_v[r, pl.ds(i, 16)] = jnp.where(
                p <= eff, g,
                jnp.where(p == eff1, jnp.int32(SEP_ID), jnp.int32(PAD_ID)))
        return 0

    couts = []
    for q in range(NQ):
        cins[q].wait()
        lax.fori_loop(q * QROWS, (q + 1) * QROWS, row_body, 0)
        couts.append(
            pltpu.async_copy(word_v.at[pl.ds(q * QROWS, QROWS)],
                             word_hbm.at[pl.ds(base + q * QROWS, QROWS)],
                             sems_out.at[q]))
    for c in couts:
        c.wait()


def _mask_body(len_ref, mask_ref):
    eff1 = jnp.minimum(len_ref[:, :], TRIM) + 1
    pos = lax.broadcasted_iota(jnp.int32, mask_ref.shape, 1)
    mask_ref[:, :] = jnp.where(pos <= eff1, jnp.int32(1), jnp.int32(0))


@jax.jit
def kernel(tokens, lengths):
    mesh = plsc.VectorSubcoreMesh(
        core_axis_name="c", subcore_axis_name="s",
        num_cores=NC, num_subcores=NS)
    words_fn = pl.kernel(
        _words_body,
        out_type=jax.ShapeDtypeStruct((B, SEQ), jnp.int32),
        mesh=mesh,
        scratch_types=[
            pltpu.VMEM((ROWS_PER_W, SEQ), jnp.int32),
            pltpu.VMEM((ROWS_PER_W, SEQ), jnp.int32),
            pltpu.VMEM((ROWS_PER_W,), jnp.int32),
            pltpu.SemaphoreType.DMA((NQ,)),
            pltpu.SemaphoreType.DMA((NQ,)),
        ],
        compiler_params=pltpu.CompilerParams(needs_layout_passes=False),
    )
    word_ids = words_fn(tokens, lengths)

    mask = pl.pallas_call(
        _mask_body,
        out_shape=jax.ShapeDtypeStruct((B, SEQ), jnp.int32),
        grid=(4,),
        in_specs=[pl.BlockSpec((B // 4, 1), lambda i: (i, 0))],
        out_specs=pl.BlockSpec((B // 4, SEQ), lambda i: (i, 0)),
    )(lengths.reshape(B, 1))
    return word_ids, mask


# speedup vs baseline: 1.3146x; 1.3146x over previous
import jax
import jax.numpy as jnp
from jax import lax
from jax.experimental import pallas as pl
from jax.experimental.pallas import tpu as pltpu
from jax.experimental.pallas import tpu_sc as plsc

SEQ = 512
CLS_ID = 101
SEP_ID = 102
PAD_ID = 0
TRIM = SEQ - 2
KCHUNKS = SEQ // 16

NC = 2
NS = 16
NW = NC * NS
B = 1024
ROWS_PER_W = B // NW
NQ = 4
QROWS = ROWS_PER_W // NQ
QITER = QROWS * KCHUNKS


def _words_body(tokens_hbm, lengths_hbm, word_hbm,
                tok_v, word_v, len_v, eff1_v, sems_in, sems_out):
    wid = lax.axis_index("s") * NC + lax.axis_index("c")
    base = wid * ROWS_PER_W

    cins = [
        pltpu.async_copy(tokens_hbm.at[pl.ds(base + q * QROWS, QROWS)],
                         tok_v.at[pl.ds(q * QROWS, QROWS)], sems_in.at[q])
        for q in range(NQ)
    ]
    pltpu.sync_copy(lengths_hbm.at[pl.ds(base, ROWS_PER_W)], len_v)

    iota16 = lax.iota(jnp.int32, 16)
    iota_m1c = jnp.maximum(iota16 - 1, 0)
    lane0 = iota16 == 0
    cls16 = jnp.full((16,), CLS_ID, jnp.int32)
    sep16 = jnp.int32(SEP_ID)
    pad16 = jnp.int32(PAD_ID)
    zero16 = jnp.full((16,), 0, jnp.int32)

    @plsc.parallel_loop(0, ROWS_PER_W, unroll=4)
    def len_body(r):
        rvec = jnp.full((16,), r, jnp.int32)
        eff1_v[r, pl.ds(0, 16)] = (
            jnp.minimum(plsc.load_gather(len_v, [rvec]), TRIM) + 1)

    couts = []
    for q in range(NQ):
        cins[q].wait()
        qrow = q * QROWS

        @plsc.parallel_loop(0, QITER, unroll=8)
        def flat_body(t):
            r = qrow + (t >> 5)
            i = (t & 31) * 16
            rvec = jnp.full((16,), r, jnp.int32)
            p = iota16 + i
            g = plsc.load_gather(tok_v, [rvec, iota_m1c + i])
            eff1 = eff1_v[r, pl.ds(0, 16)]
            word_v[r, pl.ds(i, 16)] = jnp.where(
                p < eff1, g, jnp.where(p == eff1, sep16, pad16))

        @plsc.parallel_loop(qrow, qrow + QROWS)
        def cls_body(r):
            rvec = jnp.full((16,), r, jnp.int32)
            plsc.store_scatter(word_v, [rvec, zero16], cls16, mask=lane0)

        couts.append(
            pltpu.async_copy(word_v.at[pl.ds(qrow, QROWS)],
                             word_hbm.at[pl.ds(base + qrow, QROWS)],
                             sems_out.at[q]))
    for c in couts:
        c.wait()


def _mask_body(len_ref, mask_ref):
    eff1 = jnp.minimum(len_ref[:, :], TRIM) + 1
    pos = lax.broadcasted_iota(jnp.int32, mask_ref.shape, 1)
    mask_ref[:, :] = jnp.where(pos <= eff1, jnp.int32(1), jnp.int32(0))


@jax.jit
def kernel(tokens, lengths):
    mesh = plsc.VectorSubcoreMesh(
        core_axis_name="c", subcore_axis_name="s",
        num_cores=NC, num_subcores=NS)
    words_fn = pl.kernel(
        _words_body,
        out_type=jax.ShapeDtypeStruct((B, SEQ), jnp.int32),
        mesh=mesh,
        scratch_types=[
            pltpu.VMEM((ROWS_PER_W, SEQ), jnp.int32),
            pltpu.VMEM((ROWS_PER_W, SEQ), jnp.int32),
            pltpu.VMEM((ROWS_PER_W,), jnp.int32),
            pltpu.VMEM((ROWS_PER_W, 16), jnp.int32),
            pltpu.SemaphoreType.DMA((NQ,)),
            pltpu.SemaphoreType.DMA((NQ,)),
        ],
        compiler_params=pltpu.CompilerParams(needs_layout_passes=False),
    )
    word_ids = words_fn(tokens, lengths)

    mask = pl.pallas_call(
        _mask_body,
        out_shape=jax.ShapeDtypeStruct((B, SEQ), jnp.int32),
        grid=(4,),
        in_specs=[pl.BlockSpec((B // 4, 1), lambda i: (i, 0))],
        out_specs=pl.BlockSpec((B // 4, SEQ), lambda i: (i, 0)),
    )(lengths.reshape(B, 1))
    return word_ids, mask
